# baseline (speedup 1.0000x reference)
import jax
import jax.numpy as jnp
from jax.experimental import pallas as pl

N = 512
D = 128
E_DIM = 16
H = 64
BI = 32


def _mp_block(x_blk_ref, x_full_ref, e_ref, adj_ref, w1a_ref, w1b_ref,
              w1e_ref, b1_ref, w2_ref, b2_ref, u1x_ref, u1a_ref, c1_ref,
              u2_ref, c2_ref, out_ref):
    x_blk = x_blk_ref[...]
    a = jnp.dot(x_blk, w1a_ref[...],
                preferred_element_type=jnp.float32) + b1_ref[...]
    bm = jnp.dot(x_full_ref[...], w1b_ref[...],
                 preferred_element_type=jnp.float32)
    ep = jnp.dot(e_ref[...], w1e_ref[...],
                 preferred_element_type=jnp.float32)
    pre = ep.reshape(BI, N, H) + a[:, None, :] + bm[None, :, :]
    hmsg = jnp.maximum(pre, 0.0)

    adj = adj_ref[...]
    maskf = (adj > 0).astype(jnp.float32)
    msum = jnp.sum(hmsg * maskf[:, :, None], axis=1)
    count = jnp.sum(maskf, axis=1, keepdims=True)
    deg = jnp.sum(adj, axis=1, keepdims=True)
    degf = jnp.where(deg == 0, 1, deg).astype(jnp.float32)

    agg = (jnp.dot(msum, w2_ref[...], preferred_element_type=jnp.float32)
           + count * b2_ref[...]) / degf
    hid = jnp.maximum(
        jnp.dot(x_blk, u1x_ref[...], preferred_element_type=jnp.float32)
        + jnp.dot(agg, u1a_ref[...], preferred_element_type=jnp.float32)
        + c1_ref[...], 0.0)
    out_ref[...] = (jnp.dot(hid, u2_ref[...],
                            preferred_element_type=jnp.float32)
                    + c2_ref[...])


def kernel(node_features, edge_features, adjacency, W1, b1, W2, b2, U1, c1,
           U2, c2):
    w1a = W1[:D]
    w1b = W1[D:2 * D]
    w1e = W1[2 * D:]
    u1x = U1[:D]
    u1a = U1[D:]
    b1r = b1.reshape(1, H)
    b2r = b2.reshape(1, H)
    c1r = c1.reshape(1, H)
    c2r = c2.reshape(1, H)

    grid = (N // BI,)
    full = lambda i: (0, 0)
    out = pl.pallas_call(
        _mp_block,
        grid=grid,
        in_specs=[
            pl.BlockSpec((BI, D), lambda i: (i, 0)),
            pl.BlockSpec((N, D), full),
            pl.BlockSpec((BI * N, E_DIM), lambda i: (i, 0)),
            pl.BlockSpec((BI, N), lambda i: (i, 0)),
            pl.BlockSpec((D, H), full),
            pl.BlockSpec((D, H), full),
            pl.BlockSpec((E_DIM, H), full),
            pl.BlockSpec((1, H), full),
            pl.BlockSpec((H, H), full),
            pl.BlockSpec((1, H), full),
            pl.BlockSpec((D, H), full),
            pl.BlockSpec((H, H), full),
            pl.BlockSpec((1, H), full),
            pl.BlockSpec((H, H), full),
            pl.BlockSpec((1, H), full),
        ],
        out_specs=pl.BlockSpec((BI, H), lambda i: (i, 0)),
        out_shape=jax.ShapeDtypeStruct((N, H), jnp.float32),
    )(node_features, node_features, edge_features, adjacency, w1a, w1b,
      w1e, b1r, W2, b2r, u1x, u1a, c1r, U2, c2r)
    return out
